# baseline (speedup 1.0000x reference)
import functools

import jax
import jax.numpy as jnp
from jax import lax
from jax.experimental import pallas as pl
from jax.experimental.pallas import tpu as pltpu
from jax.experimental.pallas import tpu_sc as plsc

T, N, E, D = 4, 10000, 320000, 128
HID, LAT = 128, 64
NPAD = 10240
NC, NS = 2, 16
NW = NC * NS
CH = 64
CPW = 160
EROWS = NW * CPW
EPAD = EROWS * CH - E
ACH = 128
ACPW = 80
RPS = NPAD // NS
ACCR = 10112
ARS = ACCR // NS
BLK = 1024
NBLK = NPAD // BLK

_mesh = plsc.VectorSubcoreMesh(
    core_axis_name="c", subcore_axis_name="s", num_cores=NC, num_subcores=NS)


def _fill(ref, n, value):
    for i in range(n // 16):
        ref[pl.ds(i * 16, 16)] = jnp.full((16,), value, jnp.float32)


def _deg_body(dstp, degp, deg_sh, dst2d, ones_v, zer1, sem_s):
    cid = lax.axis_index("c")
    sid = lax.axis_index("s")
    wid = sid * NC + cid
    _fill(ones_v, ACH, 1.0)
    _fill(zer1, RPS, 0.0)
    for t in range(T):
        pltpu.sync_copy(dstp.at[t, pl.ds(wid * ACPW, ACPW), :], dst2d)
        pltpu.sync_copy(zer1, deg_sh.at[pl.ds(sid * RPS, RPS)])
        plsc.subcore_barrier()

        def fire(k, carry):
            pltpu.async_copy(ones_v, deg_sh.at[dst2d.at[k]], sem_s,
                             add=True)
            return carry

        lax.fori_loop(0, ACPW, fire, 0)

        def drain(k, carry):
            pltpu.make_async_copy(ones_v, deg_sh.at[dst2d.at[0]],
                                  sem_s).wait()
            return carry

        lax.fori_loop(0, ACPW, drain, 0)
        plsc.subcore_barrier()
        pltpu.sync_copy(deg_sh.at[pl.ds(sid * RPS, RPS)],
                        degp.at[t, pl.ds(cid * NPAD + sid * RPS, RPS)])


_deg_call = pl.kernel(
    _deg_body,
    out_type=jax.ShapeDtypeStruct((T, NC * NPAD), jnp.float32),
    mesh=_mesh,
    scratch_types=[
        pltpu.VMEM_SHARED((NPAD,), jnp.float32),
        pltpu.VMEM((ACPW, ACH), jnp.int32),
        pltpu.VMEM((ACH,), jnp.float32),
        pltpu.VMEM((RPS,), jnp.float32),
        pltpu.SemaphoreType.DMA,
    ],
)


def _b_body(x_ref, degp_ref, w1_ref, g_ref, dis_ref):
    deg = degp_ref[0, 0] + degp_ref[0, 1] + 1.0
    dis = lax.rsqrt(deg)
    h0 = lax.dot_general(x_ref[0], w1_ref[...], (((1,), (1,)), ((), ())),
                         preferred_element_type=jnp.float32)
    g_ref[0] = h0 * dis
    dis_ref[0] = dis


def _run_b(xp, degp4, gW1):
    return pl.pallas_call(
        _b_body,
        grid=(T, NBLK),
        in_specs=[
            pl.BlockSpec((1, BLK, D), lambda t, j: (t, j, 0)),
            pl.BlockSpec((1, NC, BLK, 1), lambda t, j: (t, 0, j, 0)),
            pl.BlockSpec((HID, D), lambda t, j: (0, 0)),
        ],
        out_specs=[
            pl.BlockSpec((1, BLK, HID), lambda t, j: (t, j, 0)),
            pl.BlockSpec((1, BLK, 1), lambda t, j: (t, j, 0)),
        ],
        out_shape=[
            jax.ShapeDtypeStruct((T, NPAD, HID), jnp.float32),
            jax.ShapeDtypeStruct((T, NPAD, 1), jnp.float32),
        ],
    )(xp, degp4, gW1)


def _edge_body(p0, p1, p2, p3, g0, g1, g2, g3,
               q0, q1, q2, q3, zrows, accp, srcwp,
               acc_sh, srcw_sh, packed_a,
               rows0, rows1, rows2, rows3, vl0, vl1, vl2, vl3,
               sb0, sb1, sb2, sb3, db0, db1, db2, db3, zer1,
               *sems):
    cid = lax.axis_index("c")
    sid = lax.axis_index("s")
    wid = sid * NC + cid
    rows = (rows0, rows1, rows2, rows3)
    vls = (vl0, vl1, vl2, vl3)
    srcb = (sb0, sb1, sb2, sb3)
    dstb = (db0, db1, db2, db3)
    sg = sems[0:4]
    sv = sems[4:8]
    sr = sems[8:12]
    ss = sems[12:16]
    _fill(zer1, RPS, 0.0)
    for t in range(T):
        p_h = (p0, p1, p2, p3)[t]
        g_h = (g0, g1, g2, g3)[t]
        dis_h = (q0, q1, q2, q3)[t]

        def gath(k, j):
            kr = lax.rem(k, CPW // 2)
            for u in range(CH // 16):
                pk = packed_a[kr, pl.ds(u * 16, 16)]
                srcb[j][pl.ds(u * 16, 16)] = pk & 0xFFFF
                dstb[j][pl.ds(u * 16, 16)] = lax.shift_right_logical(pk, 16)
            pltpu.async_copy(g_h.at[srcb[j]], rows[j], sg[j])
            pltpu.async_copy(dis_h.at[dstb[j]], vls[j], sv[j])

        def wait_gath(j):
            pltpu.make_async_copy(g_h.at[srcb[j]], rows[j],
                                  sg[j]).wait()
            pltpu.make_async_copy(dis_h.at[dstb[j]], vls[j],
                                  sv[j]).wait()

        def scat(k, j):
            pltpu.async_copy(rows[j], acc_sh.at[dstb[j]], sr[j],
                             add=True)
            pltpu.async_copy(vls[j], srcw_sh.at[srcb[j]], ss[j],
                             add=True)

        def wait_scat(j):
            pltpu.make_async_copy(rows[j], acc_sh.at[dstb[j]],
                                  sr[j]).wait()
            pltpu.make_async_copy(vls[j], srcw_sh.at[srcb[j]],
                                  ss[j]).wait()

        pltpu.sync_copy(p_h.at[pl.ds(wid * CPW, CPW // 2), :], packed_a)
        gath(0, 0)
        gath(1, 1)
        pltpu.sync_copy(zrows.at[pl.ds(sid * ARS, ARS), :],
                        acc_sh.at[pl.ds(sid * ARS, ARS), :])
        pltpu.sync_copy(zer1, srcw_sh.at[pl.ds(sid * RPS, RPS)])
        plsc.subcore_barrier()

        def body(i, carry):
            k = 4 * i

            wait_gath(0)

            @pl.when(i > 0)
            def _():
                wait_scat(2)

            gath(k + 2, 2)
            scat(k, 0)
            wait_gath(1)

            @pl.when(i > 0)
            def _():
                wait_scat(3)

            gath(k + 3, 3)
            scat(k + 1, 1)

            @pl.when(i == CPW // 8 - 1)
            def _():
                pltpu.sync_copy(
                    p_h.at[pl.ds(wid * CPW + CPW // 2, CPW // 2), :],
                    packed_a)

            wait_gath(2)
            wait_scat(0)

            @pl.when(i < CPW // 4 - 1)
            def _():
                gath(k + 4, 0)

            scat(k + 2, 2)
            wait_gath(3)
            wait_scat(1)

            @pl.when(i < CPW // 4 - 1)
            def _():
                gath(k + 5, 1)

            scat(k + 3, 3)
            return carry

        lax.fori_loop(0, CPW // 4, body, 0)
        wait_scat(2)
        wait_scat(3)
        plsc.subcore_barrier()
        pltpu.sync_copy(acc_sh.at[pl.ds(sid * ARS, ARS), :],
                        accp.at[t, pl.ds(cid * NPAD + sid * ARS, ARS), :])
        pltpu.sync_copy(srcw_sh.at[pl.ds(sid * RPS, RPS)],
                        srcwp.at[t, pl.ds(cid * NPAD + sid * RPS, RPS)])


_edge_call = pl.kernel(
    _edge_body,
    out_type=[
        jax.ShapeDtypeStruct((T, NC * NPAD, HID), jnp.float32),
        jax.ShapeDtypeStruct((T, NC * NPAD), jnp.float32),
    ],
    mesh=_mesh,
    scratch_types=[
        pltpu.VMEM_SHARED((ACCR, HID), jnp.float32),
        pltpu.VMEM_SHARED((NPAD,), jnp.float32),
        pltpu.VMEM((CPW // 2, CH), jnp.int32),
        pltpu.VMEM((CH, HID), jnp.float32),
        pltpu.VMEM((CH, HID), jnp.float32),
        pltpu.VMEM((CH, HID), jnp.float32),
        pltpu.VMEM((CH, HID), jnp.float32),
        pltpu.VMEM((CH,), jnp.float32),
        pltpu.VMEM((CH,), jnp.float32),
        pltpu.VMEM((CH,), jnp.float32),
        pltpu.VMEM((CH,), jnp.float32),
        pltpu.VMEM((CH,), jnp.int32),
        pltpu.VMEM((CH,), jnp.int32),
        pltpu.VMEM((CH,), jnp.int32),
        pltpu.VMEM((CH,), jnp.int32),
        pltpu.VMEM((CH,), jnp.int32),
        pltpu.VMEM((CH,), jnp.int32),
        pltpu.VMEM((CH,), jnp.int32),
        pltpu.VMEM((CH,), jnp.int32),
        pltpu.VMEM((RPS,), jnp.float32),
    ] + [pltpu.SemaphoreType.DMA] * 16,
)


def _mm(a, b):
    return lax.dot_general(a, b, (((1,), (1,)), ((), ())),
                           preferred_element_type=jnp.float32)


def _d_body(g_ref, accp_ref, dis_ref, srcwp_ref, gb1_ref, gW2_ref, gb2_ref,
            Wih0_ref, Whh0_ref, b0_ref, Wih1_ref, Whh1_ref, b1_ref,
            muW1_ref, mub1_ref, muW2_ref, mub2_ref, muW3_ref, mub3_ref,
            lvW1_ref, lvb1_ref, lvW2_ref, lvb2_ref, lvW3_ref, lvb3_ref,
            mu_ref, lv_ref, pooled_ref):
    t = pl.program_id(0)
    j = pl.program_id(1)
    acc = accp_ref[0, 0] + accp_ref[0, 1]
    dis = dis_ref[0]
    idx = j * BLK + lax.broadcasted_iota(jnp.int32, (BLK, 1), 0)
    h1 = jax.nn.gelu(dis * (acc + g_ref[0]) + gb1_ref[...])
    h1 = jnp.where(idx < N, h1, 0.0)
    srcw = srcwp_ref[0, 0] + srcwp_ref[0, 1]
    w = jnp.where(idx < N, dis * srcw + dis * dis, 0.0)
    contrib = lax.dot_general(w, h1, (((0,), (0,)), ((), ())),
                              preferred_element_type=jnp.float32)

    @pl.when((t == 0) & (j == 0))
    def _init():
        pooled_ref[...] = jnp.zeros((8, HID), jnp.float32)

    onehot = lax.broadcasted_iota(jnp.int32, (8, 1), 0) == t
    pooled_ref[...] += jnp.where(onehot, jnp.broadcast_to(contrib, (8, HID)),
                                 0.0)

    @pl.when((t == T - 1) & (j == NBLK - 1))
    def _finish():
        pp = pooled_ref[0:4, :]
        pooled4 = _mm(pp, gW2_ref[...]) + float(N) * gb2_ref[...]
        h = jnp.zeros((1, HID), jnp.float32)
        c = jnp.zeros((1, HID), jnp.float32)
        ys = []
        for tt in range(T):
            gts = (_mm(pooled4[tt:tt + 1, :], Wih0_ref[...])
                   + _mm(h, Whh0_ref[...]) + b0_ref[...])
            i_ = jax.nn.sigmoid(gts[:, 0:HID])
            f_ = jax.nn.sigmoid(gts[:, HID:2 * HID])
            gg = jnp.tanh(gts[:, 2 * HID:3 * HID])
            o_ = jax.nn.sigmoid(gts[:, 3 * HID:4 * HID])
            c = f_ * c + i_ * gg
            h = o_ * jnp.tanh(c)
            ys.append(h)
        h2 = jnp.zeros((1, HID), jnp.float32)
        c2 = jnp.zeros((1, HID), jnp.float32)
        for tt in range(T):
            gts = (_mm(ys[tt], Wih1_ref[...]) + _mm(h2, Whh1_ref[...])
                   + b1_ref[...])
            i_ = jax.nn.sigmoid(gts[:, 0:HID])
            f_ = jax.nn.sigmoid(gts[:, HID:2 * HID])
            gg = jnp.tanh(gts[:, 2 * HID:3 * HID])
            o_ = jax.nn.sigmoid(gts[:, 3 * HID:4 * HID])
            c2 = f_ * c2 + i_ * gg
            h2 = o_ * jnp.tanh(c2)
        m = jax.nn.gelu(_mm(h2, muW1_ref[...]) + mub1_ref[...])
        m = jax.nn.gelu(_mm(m, muW2_ref[...]) + mub2_ref[...])
        mu_ref[...] = _mm(m, muW3_ref[...]) + mub3_ref[...]
        v = jax.nn.gelu(_mm(h2, lvW1_ref[...]) + lvb1_ref[...])
        v = jax.nn.gelu(_mm(v, lvW2_ref[...]) + lvb2_ref[...])
        lv_ref[...] = jnp.minimum(_mm(v, lvW3_ref[...]) + lvb3_ref[...], 10.0)


def _full(shape):
    nd = len(shape)
    return pl.BlockSpec(shape, lambda t, j, _n=nd: (0,) * _n)


def _run_d(g, accp4, dis3, srcwp4, weights):
    in_specs = [
        pl.BlockSpec((1, BLK, HID), lambda t, j: (t, j, 0)),
        pl.BlockSpec((1, NC, BLK, HID), lambda t, j: (t, 0, j, 0)),
        pl.BlockSpec((1, BLK, 1), lambda t, j: (t, j, 0)),
        pl.BlockSpec((1, NC, BLK, 1), lambda t, j: (t, 0, j, 0)),
    ] + [_full(w.shape) for w in weights]
    return pl.pallas_call(
        _d_body,
        grid=(T, NBLK),
        in_specs=in_specs,
        out_specs=[
            pl.BlockSpec((1, LAT), lambda t, j: (0, 0)),
            pl.BlockSpec((1, LAT), lambda t, j: (0, 0)),
        ],
        out_shape=[
            jax.ShapeDtypeStruct((1, LAT), jnp.float32),
            jax.ShapeDtypeStruct((1, LAT), jnp.float32),
        ],
        scratch_shapes=[pltpu.VMEM((8, HID), jnp.float32)],
    )(g, accp4, dis3, srcwp4, *weights)


@jax.jit
def _run(x, edge_index, gW1, gb1, gW2, gb2, Wih0, Whh0, bih0, bhh0,
         Wih1, Whh1, bih1, bhh1, muW1, mub1, muW2, mub2, muW3, mub3,
         lvW1, lvb1, lvW2, lvb2, lvW3, lvb3):
    xp = jnp.pad(x, ((0, 0), (0, NPAD - N), (0, 0)))
    pad_idx = jnp.broadcast_to(N + jnp.arange(EPAD, dtype=jnp.int32) % 224,
                               (T, EPAD))
    srcp = jnp.concatenate([edge_index[:, 0, :], pad_idx], axis=1)
    dstp = jnp.concatenate([edge_index[:, 1, :], pad_idx], axis=1)
    zpad = jnp.zeros((T, EPAD), jnp.int32)
    dstc = jnp.concatenate([edge_index[:, 1, :], zpad], axis=1)
    packed = srcp | (dstc << 16)
    pks = [packed[t].reshape(EROWS, CH) for t in range(T)]
    dstp_a = dstp.reshape(T, EROWS // 2, 2 * CH)

    degp = _deg_call(dstp_a)
    degp4 = degp.reshape(T, NC, NPAD, 1)

    g, dis3 = _run_b(xp, degp4, gW1)
    gs = [g[t] for t in range(T)]
    qs = [dis3[t].reshape(NPAD) for t in range(T)]

    zrows = jnp.zeros((NPAD, HID), jnp.float32)
    accp, srcwp = _edge_call(*pks, *gs, *qs, zrows)
    accp4 = accp.reshape(T, NC, NPAD, HID)
    srcwp4 = srcwp.reshape(T, NC, NPAD, 1)

    weights = [
        gb1.reshape(1, HID), gW2, gb2.reshape(1, HID),
        Wih0, Whh0, (bih0 + bhh0).reshape(1, 4 * HID),
        Wih1, Whh1, (bih1 + bhh1).reshape(1, 4 * HID),
        muW1, mub1.reshape(1, HID), muW2, mub2.reshape(1, HID),
        muW3, mub3.reshape(1, LAT),
        lvW1, lvb1.reshape(1, HID), lvW2, lvb2.reshape(1, HID),
        lvW3, lvb3.reshape(1, LAT),
    ]
    mu, lv = _run_d(g, accp4, dis3, srcwp4, weights)
    return (mu, lv, mu)


def kernel(x, edge_index, gW1, gb1, gW2, gb2, Wih0, Whh0, bih0, bhh0,
           Wih1, Whh1, bih1, bhh1, muW1, mub1, muW2, mub2, muW3, mub3,
           lvW1, lvb1, lvW2, lvb2, lvW3, lvb3):
    return _run(x, edge_index, gW1, gb1, gW2, gb2, Wih0, Whh0, bih0, bhh0,
                Wih1, Whh1, bih1, bhh1, muW1, mub1, muW2, mub2, muW3, mub3,
                lvW1, lvb1, lvW2, lvb2, lvW3, lvb3)


# speedup vs baseline: 1.0245x; 1.0245x over previous
import functools

import jax
import jax.numpy as jnp
from jax import lax
from jax.experimental import pallas as pl
from jax.experimental.pallas import tpu as pltpu
from jax.experimental.pallas import tpu_sc as plsc

T, N, E, D = 4, 10000, 320000, 128
HID, LAT = 128, 64
NPAD = 10240
NC, NS = 2, 16
NW = NC * NS
CH = 64
CPW = 160
EROWS = NW * CPW
EPAD = EROWS * CH - E
ACH = 128
ACPW = 80
RPS = NPAD // NS
ACCR = 10112
ARS = ACCR // NS
BLK = 1024
NBLK = NPAD // BLK

_mesh = plsc.VectorSubcoreMesh(
    core_axis_name="c", subcore_axis_name="s", num_cores=NC, num_subcores=NS)


def _fill(ref, n, value):
    for i in range(n // 16):
        ref[pl.ds(i * 16, 16)] = jnp.full((16,), value, jnp.float32)


def _deg_body(dstp, degp, deg_sh, dst2d, ones_v, zer1, sem_s):
    cid = lax.axis_index("c")
    sid = lax.axis_index("s")
    wid = sid * NC + cid
    _fill(ones_v, ACH, 1.0)
    _fill(zer1, RPS, 0.0)
    for t in range(T):
        pltpu.sync_copy(dstp.at[t, pl.ds(wid * ACPW, ACPW), :], dst2d)
        pltpu.sync_copy(zer1, deg_sh.at[pl.ds(sid * RPS, RPS)])
        plsc.subcore_barrier()

        def fire(k, carry):
            pltpu.async_copy(ones_v, deg_sh.at[dst2d.at[k]], sem_s,
                             add=True)
            return carry

        lax.fori_loop(0, ACPW, fire, 0)

        def drain(k, carry):
            pltpu.make_async_copy(ones_v, deg_sh.at[dst2d.at[0]],
                                  sem_s).wait()
            return carry

        lax.fori_loop(0, ACPW, drain, 0)
        plsc.subcore_barrier()
        pltpu.sync_copy(deg_sh.at[pl.ds(sid * RPS, RPS)],
                        degp.at[t, pl.ds(cid * NPAD + sid * RPS, RPS)])


_deg_call = pl.kernel(
    _deg_body,
    out_type=jax.ShapeDtypeStruct((T, NC * NPAD), jnp.float32),
    mesh=_mesh,
    scratch_types=[
        pltpu.VMEM_SHARED((NPAD,), jnp.float32),
        pltpu.VMEM((ACPW, ACH), jnp.int32),
        pltpu.VMEM((ACH,), jnp.float32),
        pltpu.VMEM((RPS,), jnp.float32),
        pltpu.SemaphoreType.DMA,
    ],
)


def _b_body(x_ref, degp_ref, w1_ref, g_ref, dis_ref):
    deg = degp_ref[0, 0] + degp_ref[0, 1] + 1.0
    dis = lax.rsqrt(deg)
    h0 = lax.dot_general(x_ref[0], w1_ref[...], (((1,), (1,)), ((), ())),
                         preferred_element_type=jnp.float32)
    g_ref[0] = h0 * dis
    dis_ref[0] = dis


def _run_b(xp, degp4, gW1):
    return pl.pallas_call(
        _b_body,
        grid=(T, NBLK),
        in_specs=[
            pl.BlockSpec((1, BLK, D), lambda t, j: (t, j, 0)),
            pl.BlockSpec((1, NC, BLK, 1), lambda t, j: (t, 0, j, 0)),
            pl.BlockSpec((HID, D), lambda t, j: (0, 0)),
        ],
        out_specs=[
            pl.BlockSpec((1, BLK, HID), lambda t, j: (t, j, 0)),
            pl.BlockSpec((1, BLK, 1), lambda t, j: (t, j, 0)),
        ],
        out_shape=[
            jax.ShapeDtypeStruct((T, NPAD, HID), jnp.float32),
            jax.ShapeDtypeStruct((T, NPAD, 1), jnp.float32),
        ],
    )(xp, degp4, gW1)


def _edge_body(p0, p1, p2, p3, g0, g1, g2, g3,
               q0, q1, q2, q3, zrows, accp, srcwp,
               acc_sh, srcw_sh, packed_a,
               rows0, rows1, rows2, rows3, vl0, vl1, vl2, vl3,
               sb0, sb1, sb2, sb3, db0, db1, db2, db3, zer1,
               *sems):
    cid = lax.axis_index("c")
    sid = lax.axis_index("s")
    wid = sid * NC + cid
    rows = (rows0, rows1, rows2, rows3)
    vls = (vl0, vl1, vl2, vl3)
    srcb = (sb0, sb1, sb2, sb3)
    dstb = (db0, db1, db2, db3)
    sg = sems[0:4]
    sv = sems[4:8]
    sr = sems[8:12]
    ss = sems[12:16]
    sw = sems[16]
    _fill(zer1, RPS, 0.0)
    for t in range(T):
        p_h = (p0, p1, p2, p3)[t]
        g_h = (g0, g1, g2, g3)[t]
        dis_h = (q0, q1, q2, q3)[t]

        def gath(k, j):
            kr = lax.rem(k, CPW // 2)
            for u in range(CH // 16):
                pk = packed_a[kr, pl.ds(u * 16, 16)]
                srcb[j][pl.ds(u * 16, 16)] = pk & 0xFFFF
                dstb[j][pl.ds(u * 16, 16)] = lax.shift_right_logical(pk, 16)
            pltpu.async_copy(g_h.at[srcb[j]], rows[j], sg[j])
            pltpu.async_copy(dis_h.at[dstb[j]], vls[j], sv[j])

        def wait_gath(j):
            pltpu.make_async_copy(g_h.at[srcb[j]], rows[j],
                                  sg[j]).wait()
            pltpu.make_async_copy(dis_h.at[dstb[j]], vls[j],
                                  sv[j]).wait()

        def scat(k, j):
            pltpu.async_copy(rows[j], acc_sh.at[dstb[j]], sr[j],
                             add=True)
            pltpu.async_copy(vls[j], srcw_sh.at[srcb[j]], ss[j],
                             add=True)

        def wait_scat(j):
            pltpu.make_async_copy(rows[j], acc_sh.at[dstb[j]],
                                  sr[j]).wait()
            pltpu.make_async_copy(vls[j], srcw_sh.at[srcb[j]],
                                  ss[j]).wait()

        pltpu.sync_copy(p_h.at[pl.ds(wid * CPW, CPW // 2), :], packed_a)
        gath(0, 0)
        gath(1, 1)
        if t > 0:
            pltpu.make_async_copy(
                acc_sh.at[pl.ds(sid * ARS, ARS), :],
                accp.at[t - 1, pl.ds(cid * NPAD + sid * ARS, ARS), :],
                sw).wait()
            pltpu.make_async_copy(
                srcw_sh.at[pl.ds(sid * RPS, RPS)],
                srcwp.at[t - 1, pl.ds(cid * NPAD + sid * RPS, RPS)],
                sw).wait()
        pltpu.sync_copy(zrows.at[pl.ds(sid * ARS, ARS), :],
                        acc_sh.at[pl.ds(sid * ARS, ARS), :])
        pltpu.sync_copy(zer1, srcw_sh.at[pl.ds(sid * RPS, RPS)])
        plsc.subcore_barrier()

        def body(i, carry):
            k = 4 * i

            @pl.when(i > 0)
            def _():
                wait_scat(2)
                wait_scat(3)

            gath(k + 2, 2)
            gath(k + 3, 3)
            wait_gath(0)
            scat(k, 0)
            wait_gath(1)
            scat(k + 1, 1)
            wait_scat(0)
            wait_scat(1)

            @pl.when(i == CPW // 8 - 1)
            def _():
                pltpu.sync_copy(
                    p_h.at[pl.ds(wid * CPW + CPW // 2, CPW // 2), :],
                    packed_a)

            @pl.when(i < CPW // 4 - 1)
            def _():
                gath(k + 4, 0)
                gath(k + 5, 1)

            wait_gath(2)
            scat(k + 2, 2)
            wait_gath(3)
            scat(k + 3, 3)
            return carry

        lax.fori_loop(0, CPW // 4, body, 0)
        wait_scat(2)
        wait_scat(3)
        plsc.subcore_barrier()
        pltpu.async_copy(acc_sh.at[pl.ds(sid * ARS, ARS), :],
                         accp.at[t, pl.ds(cid * NPAD + sid * ARS, ARS), :],
                         sw)
        pltpu.async_copy(srcw_sh.at[pl.ds(sid * RPS, RPS)],
                         srcwp.at[t, pl.ds(cid * NPAD + sid * RPS, RPS)],
                         sw)
        if t == T - 1:
            pltpu.make_async_copy(
                acc_sh.at[pl.ds(sid * ARS, ARS), :],
                accp.at[t, pl.ds(cid * NPAD + sid * ARS, ARS), :],
                sw).wait()
            pltpu.make_async_copy(
                srcw_sh.at[pl.ds(sid * RPS, RPS)],
                srcwp.at[t, pl.ds(cid * NPAD + sid * RPS, RPS)],
                sw).wait()


_edge_call = pl.kernel(
    _edge_body,
    out_type=[
        jax.ShapeDtypeStruct((T, NC * NPAD, HID), jnp.float32),
        jax.ShapeDtypeStruct((T, NC * NPAD), jnp.float32),
    ],
    mesh=_mesh,
    scratch_types=[
        pltpu.VMEM_SHARED((ACCR, HID), jnp.float32),
        pltpu.VMEM_SHARED((NPAD,), jnp.float32),
        pltpu.VMEM((CPW // 2, CH), jnp.int32),
        pltpu.VMEM((CH, HID), jnp.float32),
        pltpu.VMEM((CH, HID), jnp.float32),
        pltpu.VMEM((CH, HID), jnp.float32),
        pltpu.VMEM((CH, HID), jnp.float32),
        pltpu.VMEM((CH,), jnp.float32),
        pltpu.VMEM((CH,), jnp.float32),
        pltpu.VMEM((CH,), jnp.float32),
        pltpu.VMEM((CH,), jnp.float32),
        pltpu.VMEM((CH,), jnp.int32),
        pltpu.VMEM((CH,), jnp.int32),
        pltpu.VMEM((CH,), jnp.int32),
        pltpu.VMEM((CH,), jnp.int32),
        pltpu.VMEM((CH,), jnp.int32),
        pltpu.VMEM((CH,), jnp.int32),
        pltpu.VMEM((CH,), jnp.int32),
        pltpu.VMEM((CH,), jnp.int32),
        pltpu.VMEM((RPS,), jnp.float32),
    ] + [pltpu.SemaphoreType.DMA] * 17,
)


def _mm(a, b):
    return lax.dot_general(a, b, (((1,), (1,)), ((), ())),
                           preferred_element_type=jnp.float32)


def _d_body(g_ref, accp_ref, dis_ref, srcwp_ref, gb1_ref, gW2_ref, gb2_ref,
            Wih0_ref, Whh0_ref, b0_ref, Wih1_ref, Whh1_ref, b1_ref,
            muW1_ref, mub1_ref, muW2_ref, mub2_ref, muW3_ref, mub3_ref,
            lvW1_ref, lvb1_ref, lvW2_ref, lvb2_ref, lvW3_ref, lvb3_ref,
            mu_ref, lv_ref, pooled_ref):
    t = pl.program_id(0)
    j = pl.program_id(1)
    acc = accp_ref[0, 0] + accp_ref[0, 1]
    dis = dis_ref[0]
    idx = j * BLK + lax.broadcasted_iota(jnp.int32, (BLK, 1), 0)
    h1 = jax.nn.gelu(dis * (acc + g_ref[0]) + gb1_ref[...])
    h1 = jnp.where(idx < N, h1, 0.0)
    srcw = srcwp_ref[0, 0] + srcwp_ref[0, 1]
    w = jnp.where(idx < N, dis * srcw + dis * dis, 0.0)
    contrib = lax.dot_general(w, h1, (((0,), (0,)), ((), ())),
                              preferred_element_type=jnp.float32)

    @pl.when((t == 0) & (j == 0))
    def _init():
        pooled_ref[...] = jnp.zeros((8, HID), jnp.float32)

    onehot = lax.broadcasted_iota(jnp.int32, (8, 1), 0) == t
    pooled_ref[...] += jnp.where(onehot, jnp.broadcast_to(contrib, (8, HID)),
                                 0.0)

    @pl.when((t == T - 1) & (j == NBLK - 1))
    def _finish():
        pp = pooled_ref[0:4, :]
        pooled4 = _mm(pp, gW2_ref[...]) + float(N) * gb2_ref[...]
        h = jnp.zeros((1, HID), jnp.float32)
        c = jnp.zeros((1, HID), jnp.float32)
        ys = []
        for tt in range(T):
            gts = (_mm(pooled4[tt:tt + 1, :], Wih0_ref[...])
                   + _mm(h, Whh0_ref[...]) + b0_ref[...])
            i_ = jax.nn.sigmoid(gts[:, 0:HID])
            f_ = jax.nn.sigmoid(gts[:, HID:2 * HID])
            gg = jnp.tanh(gts[:, 2 * HID:3 * HID])
            o_ = jax.nn.sigmoid(gts[:, 3 * HID:4 * HID])
            c = f_ * c + i_ * gg
            h = o_ * jnp.tanh(c)
            ys.append(h)
        h2 = jnp.zeros((1, HID), jnp.float32)
        c2 = jnp.zeros((1, HID), jnp.float32)
        for tt in range(T):
            gts = (_mm(ys[tt], Wih1_ref[...]) + _mm(h2, Whh1_ref[...])
                   + b1_ref[...])
            i_ = jax.nn.sigmoid(gts[:, 0:HID])
            f_ = jax.nn.sigmoid(gts[:, HID:2 * HID])
            gg = jnp.tanh(gts[:, 2 * HID:3 * HID])
            o_ = jax.nn.sigmoid(gts[:, 3 * HID:4 * HID])
            c2 = f_ * c2 + i_ * gg
            h2 = o_ * jnp.tanh(c2)
        m = jax.nn.gelu(_mm(h2, muW1_ref[...]) + mub1_ref[...])
        m = jax.nn.gelu(_mm(m, muW2_ref[...]) + mub2_ref[...])
        mu_ref[...] = _mm(m, muW3_ref[...]) + mub3_ref[...]
        v = jax.nn.gelu(_mm(h2, lvW1_ref[...]) + lvb1_ref[...])
        v = jax.nn.gelu(_mm(v, lvW2_ref[...]) + lvb2_ref[...])
        lv_ref[...] = jnp.minimum(_mm(v, lvW3_ref[...]) + lvb3_ref[...], 10.0)


def _full(shape):
    nd = len(shape)
    return pl.BlockSpec(shape, lambda t, j, _n=nd: (0,) * _n)


def _run_d(g, accp4, dis3, srcwp4, weights):
    in_specs = [
        pl.BlockSpec((1, BLK, HID), lambda t, j: (t, j, 0)),
        pl.BlockSpec((1, NC, BLK, HID), lambda t, j: (t, 0, j, 0)),
        pl.BlockSpec((1, BLK, 1), lambda t, j: (t, j, 0)),
        pl.BlockSpec((1, NC, BLK, 1), lambda t, j: (t, 0, j, 0)),
    ] + [_full(w.shape) for w in weights]
    return pl.pallas_call(
        _d_body,
        grid=(T, NBLK),
        in_specs=in_specs,
        out_specs=[
            pl.BlockSpec((1, LAT), lambda t, j: (0, 0)),
            pl.BlockSpec((1, LAT), lambda t, j: (0, 0)),
        ],
        out_shape=[
            jax.ShapeDtypeStruct((1, LAT), jnp.float32),
            jax.ShapeDtypeStruct((1, LAT), jnp.float32),
        ],
        scratch_shapes=[pltpu.VMEM((8, HID), jnp.float32)],
    )(g, accp4, dis3, srcwp4, *weights)


@jax.jit
def _run(x, edge_index, gW1, gb1, gW2, gb2, Wih0, Whh0, bih0, bhh0,
         Wih1, Whh1, bih1, bhh1, muW1, mub1, muW2, mub2, muW3, mub3,
         lvW1, lvb1, lvW2, lvb2, lvW3, lvb3):
    xp = jnp.pad(x, ((0, 0), (0, NPAD - N), (0, 0)))
    pad_idx = jnp.broadcast_to(N + jnp.arange(EPAD, dtype=jnp.int32) % 224,
                               (T, EPAD))
    srcp = jnp.concatenate([edge_index[:, 0, :], pad_idx], axis=1)
    dstp = jnp.concatenate([edge_index[:, 1, :], pad_idx], axis=1)
    zpad = jnp.zeros((T, EPAD), jnp.int32)
    dstc = jnp.concatenate([edge_index[:, 1, :], zpad], axis=1)
    packed = srcp | (dstc << 16)
    pks = [packed[t].reshape(EROWS, CH) for t in range(T)]
    dstp_a = dstp.reshape(T, EROWS // 2, 2 * CH)

    degp = _deg_call(dstp_a)
    degp4 = degp.reshape(T, NC, NPAD, 1)

    g, dis3 = _run_b(xp, degp4, gW1)
    gs = [g[t] for t in range(T)]
    qs = [dis3[t].reshape(NPAD) for t in range(T)]

    zrows = jnp.zeros((NPAD, HID), jnp.float32)
    accp, srcwp = _edge_call(*pks, *gs, *qs, zrows)
    accp4 = accp.reshape(T, NC, NPAD, HID)
    srcwp4 = srcwp.reshape(T, NC, NPAD, 1)

    weights = [
        gb1.reshape(1, HID), gW2, gb2.reshape(1, HID),
        Wih0, Whh0, (bih0 + bhh0).reshape(1, 4 * HID),
        Wih1, Whh1, (bih1 + bhh1).reshape(1, 4 * HID),
        muW1, mub1.reshape(1, HID), muW2, mub2.reshape(1, HID),
        muW3, mub3.reshape(1, LAT),
        lvW1, lvb1.reshape(1, HID), lvW2, lvb2.reshape(1, HID),
        lvW3, lvb3.reshape(1, LAT),
    ]
    mu, lv = _run_d(g, accp4, dis3, srcwp4, weights)
    return (mu, lv, mu)


def kernel(x, edge_index, gW1, gb1, gW2, gb2, Wih0, Whh0, bih0, bhh0,
           Wih1, Whh1, bih1, bhh1, muW1, mub1, muW2, mub2, muW3, mub3,
           lvW1, lvb1, lvW2, lvb2, lvW3, lvb3):
    return _run(x, edge_index, gW1, gb1, gW2, gb2, Wih0, Whh0, bih0, bhh0,
                Wih1, Whh1, bih1, bhh1, muW1, mub1, muW2, mub2, muW3, mub3,
                lvW1, lvb1, lvW2, lvb2, lvW3, lvb3)
